# baseline (speedup 1.0000x reference)
import jax
import jax.numpy as jnp
from jax.experimental import pallas as pl
from jax.experimental.pallas import tpu as pltpu

_CHUNK_FRACS = (512, 3584, 3584, 512)


def _make_body(seq_len, dim):
    scale = seq_len // sum(_CHUNK_FRACS)
    sizes = [c * scale for c in _CHUNK_FRACS]
    starts = [sum(sizes[:i]) for i in range(len(sizes))]
    n = len(sizes)

    def body(emb_hbm, o_hbm, scratch, in_sems, out_sems):
        reads = [
            pltpu.make_async_copy(
                emb_hbm.at[pl.ds(starts[i], sizes[i]), :],
                scratch.at[pl.ds(starts[i], sizes[i]), :],
                in_sems.at[i],
            )
            for i in range(n)
        ]
        writes = [
            pltpu.make_async_copy(
                scratch.at[pl.ds(starts[i], sizes[i]), :],
                o_hbm.at[pl.ds(starts[i], sizes[i]), :],
                out_sems.at[i],
            )
            for i in range(n)
        ]
        for r in reads:
            r.start()
        for r, w in zip(reads, writes):
            r.wait()
            w.start()
        for w in writes:
            w.wait()

    return body


def kernel(x, emb):
    seq_len = x.shape[1]
    dim = emb.shape[1]
    return pl.pallas_call(
        _make_body(seq_len, dim),
        in_specs=[pl.BlockSpec(memory_space=pl.ANY)],
        out_specs=pl.BlockSpec(memory_space=pl.ANY),
        out_shape=jax.ShapeDtypeStruct((seq_len, dim), emb.dtype),
        scratch_shapes=[
            pltpu.VMEM((seq_len, dim), emb.dtype),
            pltpu.SemaphoreType.DMA((len(_CHUNK_FRACS),)),
            pltpu.SemaphoreType.DMA((len(_CHUNK_FRACS),)),
        ],
    )(emb)


# speedup vs baseline: 1.0586x; 1.0586x over previous
import jax
import jax.numpy as jnp
from jax.experimental import pallas as pl
from jax.experimental.pallas import tpu as pltpu

_NUM_CHUNKS = 4


def _make_body(seq_len, dim):
    chunk = seq_len // _NUM_CHUNKS

    def body(emb_hbm, o_hbm, scratch, in_sems, out_sems):
        reads = [
            pltpu.make_async_copy(
                emb_hbm.at[pl.ds(i * chunk, chunk), :],
                scratch.at[pl.ds(i * chunk, chunk), :],
                in_sems.at[i],
            )
            for i in range(_NUM_CHUNKS)
        ]
        writes = [
            pltpu.make_async_copy(
                scratch.at[pl.ds(i * chunk, chunk), :],
                o_hbm.at[pl.ds(i * chunk, chunk), :],
                out_sems.at[i],
            )
            for i in range(_NUM_CHUNKS)
        ]
        for r in reads:
            r.start()
        for r, w in zip(reads, writes):
            r.wait()
            w.start()
        for w in writes:
            w.wait()

    return body


def kernel(x, emb):
    seq_len = x.shape[1]
    dim = emb.shape[1]
    return pl.pallas_call(
        _make_body(seq_len, dim),
        in_specs=[pl.BlockSpec(memory_space=pl.ANY)],
        out_specs=pl.BlockSpec(memory_space=pl.ANY),
        out_shape=jax.ShapeDtypeStruct((seq_len, dim), emb.dtype),
        scratch_shapes=[
            pltpu.VMEM((seq_len, dim), emb.dtype),
            pltpu.SemaphoreType.DMA((_NUM_CHUNKS,)),
            pltpu.SemaphoreType.DMA((_NUM_CHUNKS,)),
        ],
    )(emb)
